# baseline (speedup 1.0000x reference)
import functools

import jax
import jax.numpy as jnp
from jax import lax
from jax.experimental import pallas as pl
from jax.experimental.pallas import tpu as pltpu
from jax.experimental.pallas import tpu_sc as plsc

EMBED_DIM = 64
FEAT_DIM = 64
BATCH = 16384
H1 = 32
H2 = 16

_HALF_B = BATCH // 2
_PACK_ROWS = _HALF_B // 2

_SC_INFO = plsc.get_sparse_core_info()
_NC = _SC_INFO.num_cores
_NS = _SC_INFO.num_subcores
_NW = _NC * _NS
_B_PER_W = _HALF_B // _NW
_G = _B_PER_W // 2

_sc_mesh = plsc.VectorSubcoreMesh(core_axis_name="c", subcore_axis_name="s")


@functools.partial(
    pl.kernel,
    mesh=_sc_mesh,
    out_type=jax.ShapeDtypeStruct((_PACK_ROWS, 2 * EMBED_DIM), jnp.float32),
    scratch_types=[
        pltpu.VMEM((_G,), jnp.int32),
        pltpu.VMEM((_G,), jnp.int32),
        pltpu.VMEM((_G, EMBED_DIM), jnp.float32),
        pltpu.VMEM((_G, EMBED_DIM), jnp.float32),
        pltpu.SemaphoreType.DMA,
        pltpu.SemaphoreType.DMA,
    ],
    compiler_params=pltpu.CompilerParams(use_tc_tiling_on_sc=False),
)
def _sc_gather(table_hbm, idx_hbm, out_hbm, idx_a, idx_b, rows_a, rows_b,
               sem_a, sem_b):
    wid = lax.axis_index("s") * _NC + lax.axis_index("c")
    base_a = wid * _G
    base_b = _PACK_ROWS + base_a
    pltpu.sync_copy(idx_hbm.at[pl.ds(base_a, _G)], idx_a)
    pltpu.sync_copy(idx_hbm.at[pl.ds(base_b, _G)], idx_b)
    cp_a = pltpu.async_copy(table_hbm.at[idx_a], rows_a, sem_a)
    cp_b = pltpu.async_copy(table_hbm.at[idx_b], rows_b, sem_b)
    cp_a.wait()
    cp_b.wait()
    pltpu.sync_copy(rows_a,
                    out_hbm.at[pl.ds(base_a, _G), pl.ds(0, EMBED_DIM)])
    pltpu.sync_copy(rows_b,
                    out_hbm.at[pl.ds(base_a, _G), pl.ds(EMBED_DIM, EMBED_DIM)])


def _mlp_body(uft_ref, emb_ref, w1t_ref, b1_ref, w2t_ref, b2_ref,
              wctt_ref, wcbt_ref, bc_ref, out_ref):
    f32 = jnp.float32
    h = lax.dot_general(w1t_ref[...], uft_ref[...], (((1,), (0,)), ((), ())),
                        preferred_element_type=f32)
    h = jnp.maximum(h + b1_ref[...], 0.0)
    h = lax.dot_general(w2t_ref[...], h, (((1,), (0,)), ((), ())),
                        preferred_element_type=f32)
    h = jnp.maximum(h + b2_ref[...], 0.0)
    emb = emb_ref[...]
    y_lo = lax.dot_general(wctt_ref[...], emb[:, :EMBED_DIM],
                           (((1,), (1,)), ((), ())),
                           preferred_element_type=f32)
    y_hi = lax.dot_general(wctt_ref[...], emb[:, EMBED_DIM:],
                           (((1,), (1,)), ((), ())),
                           preferred_element_type=f32)
    y = jnp.concatenate([y_lo, y_hi], axis=1)
    y = y + lax.dot_general(wcbt_ref[...], h, (((1,), (0,)), ((), ())),
                            preferred_element_type=f32)
    out_ref[...] = jnp.maximum(y + bc_ref[...], 0.0)


def _mlp_first_body(*args):
    _mlp_body(*args)


def _mlp_second_body(uft_ref, emb_ref, w1t_ref, b1_ref, w2t_ref, b2_ref,
                     wctt_ref, wcbt_ref, bc_ref, prev_ref, out_ref):
    del prev_ref
    _mlp_body(uft_ref, emb_ref, w1t_ref, b1_ref, w2t_ref, b2_ref,
              wctt_ref, wcbt_ref, bc_ref, out_ref)


_WEIGHT_SPECS = [
    pl.BlockSpec((H1, FEAT_DIM), lambda i: (0, 0)),
    pl.BlockSpec((H1, 1), lambda i: (0, 0)),
    pl.BlockSpec((H2, H1), lambda i: (0, 0)),
    pl.BlockSpec((H2, 1), lambda i: (0, 0)),
    pl.BlockSpec((EMBED_DIM, EMBED_DIM), lambda i: (0, 0)),
    pl.BlockSpec((EMBED_DIM, H2), lambda i: (0, 0)),
    pl.BlockSpec((EMBED_DIM, 1), lambda i: (0, 0)),
]


def _mlp_first(uft, emb_p, *weights):
    return pl.pallas_call(
        _mlp_first_body,
        grid=(1,),
        in_specs=[
            pl.BlockSpec((FEAT_DIM, _HALF_B), lambda i: (0, 0)),
            pl.BlockSpec((_PACK_ROWS, 2 * EMBED_DIM), lambda i: (0, 0)),
            *_WEIGHT_SPECS,
        ],
        out_specs=pl.BlockSpec((EMBED_DIM, _HALF_B), lambda i: (0, 0)),
        out_shape=jax.ShapeDtypeStruct((EMBED_DIM, BATCH), jnp.float32),
    )(uft, emb_p, *weights)


def _mlp_second(uft, emb_p, *weights_and_prev):
    return pl.pallas_call(
        _mlp_second_body,
        grid=(1,),
        in_specs=[
            pl.BlockSpec((FEAT_DIM, _HALF_B), lambda i: (0, 1)),
            pl.BlockSpec((_PACK_ROWS, 2 * EMBED_DIM), lambda i: (0, 0)),
            *_WEIGHT_SPECS,
            pl.BlockSpec((8, 128), lambda i: (0, 0)),
        ],
        out_specs=pl.BlockSpec((EMBED_DIM, _HALF_B), lambda i: (0, 1)),
        out_shape=jax.ShapeDtypeStruct((EMBED_DIM, BATCH), jnp.float32),
        input_output_aliases={9: 0},
    )(uft, emb_p, *weights_and_prev)


def kernel(user_id, user_features, table, W1, b1, W2, b2, Wc, bc):
    idx = user_id.astype(jnp.int32)
    emb1 = _sc_gather(table, idx[:_HALF_B])
    emb2 = _sc_gather(table, idx[_HALF_B:])
    weights = (W1.T, b1.reshape(H1, 1), W2.T, b2.reshape(H2, 1),
               Wc[:EMBED_DIM].T, Wc[EMBED_DIM:].T, bc.reshape(EMBED_DIM, 1))
    uft = user_features.T
    o1 = _mlp_first(uft, emb1, *weights)
    outT = _mlp_second(uft, emb2, *weights, o1)
    return outT.T
